# baseline (speedup 1.0000x reference)
import jax
import jax.numpy as jnp
from jax.experimental import pallas as pl
from jax.experimental.pallas import tpu as pltpu

_B = 64
_V = 100000
_BLK = 8192

_K0 = 0
_K1 = 42
_K2 = _K0 ^ _K1 ^ 0x1BD11BDA

_ROTS = ((13, 15, 26, 6), (17, 29, 16, 24))
_SCHED = ((_K1, _K2), (_K2, _K0), (_K0, _K1), (_K1, _K2), (_K2, _K0))


def _threefry_bits(x1):
    x1 = x1 + jnp.uint32(_K1)
    x0 = x1
    x1 = ((x1 << jnp.uint32(13)) | (x1 >> jnp.uint32(19))) ^ x0
    first = True
    for i in range(5):
        for r in _ROTS[i % 2]:
            if first:
                first = False
                continue
            x0 = x0 + x1
            x1 = (x1 << jnp.uint32(r)) | (x1 >> jnp.uint32(32 - r))
            x1 = x1 ^ x0
        ka, kb = _SCHED[i]
        x0 = x0 + jnp.uint32(ka)
        x1 = x1 + jnp.uint32(kb) + jnp.uint32(i + 1)
    return x0 ^ x1


def _sample_kernel(x_ref, o_ref, max_ref, idx_ref):
    b = pl.program_id(0)
    nb = pl.num_programs(0)

    @pl.when(b == 0)
    def _init():
        max_ref[...] = jnp.full_like(max_ref[...], -jnp.inf)
        idx_ref[...] = jnp.zeros_like(idx_ref[...])

    shp = (_B, _BLK)
    col = jax.lax.broadcasted_iota(jnp.int32, shp, 1) + b * _BLK
    row = jax.lax.broadcasted_iota(jnp.uint32, shp, 0)
    flat = row * jnp.uint32(_V) + col.astype(jnp.uint32)

    bits = _threefry_bits(flat)

    fbits = (bits >> jnp.uint32(9)) | jnp.uint32(0x3F800000)
    u = jax.lax.bitcast_convert_type(fbits, jnp.float32) - 1.0
    u = jnp.maximum(u, jnp.float32(1e-20))
    gumbel = -jnp.log(-jnp.log(u))

    pert = x_ref[...] + gumbel
    pert = jnp.where(col < _V, pert, -jnp.inf)

    acc_m = max_ref[...]
    acc_i = idx_ref[...]
    for s in range(_BLK // 128):
        tile = pert[:, s * 128:(s + 1) * 128]
        itile = col[:, s * 128:(s + 1) * 128]
        upd = tile > acc_m
        acc_i = jnp.where(upd, itile, acc_i)
        acc_m = jnp.maximum(acc_m, tile)
    max_ref[...] = acc_m
    idx_ref[...] = acc_i

    @pl.when(b == nb - 1)
    def _done():
        m = jnp.max(acc_m, axis=1, keepdims=True)
        cand = jnp.where(acc_m == m, acc_i, jnp.int32(0x7FFFFFFF))
        o_ref[...] = jnp.min(cand, axis=1, keepdims=True)


def kernel(logits):
    out = pl.pallas_call(
        _sample_kernel,
        grid=(pl.cdiv(_V, _BLK),),
        in_specs=[pl.BlockSpec((_B, _BLK), lambda b: (0, b))],
        out_specs=pl.BlockSpec((_B, 1), lambda b: (0, 0)),
        out_shape=jax.ShapeDtypeStruct((_B, 1), jnp.int32),
        scratch_shapes=[
            pltpu.VMEM((_B, 128), jnp.float32),
            pltpu.VMEM((_B, 128), jnp.int32),
        ],
    )(logits)
    return out[:, 0].astype(jnp.int64)


# speedup vs baseline: 1.3689x; 1.3689x over previous
import jax
import jax.numpy as jnp
from jax.experimental import pallas as pl
from jax.experimental.pallas import tpu as pltpu

_B = 64
_V = 100000
_BLK = 8192
_CHUNK = 2048

_K0 = 0
_K1 = 42
_K2 = _K0 ^ _K1 ^ 0x1BD11BDA

_ROTS = ((13, 15, 26, 6), (17, 29, 16, 24))
_SCHED = ((_K1, _K2), (_K2, _K0), (_K0, _K1), (_K1, _K2), (_K2, _K0))


def _threefry_bits(x1):
    x1 = x1 + jnp.uint32(_K1)
    x0 = x1
    x1 = ((x1 << jnp.uint32(13)) | (x1 >> jnp.uint32(19))) ^ x0
    first = True
    for i in range(5):
        for r in _ROTS[i % 2]:
            if first:
                first = False
                continue
            x0 = x0 + x1
            x1 = (x1 << jnp.uint32(r)) | (x1 >> jnp.uint32(32 - r))
            x1 = x1 ^ x0
        ka, kb = _SCHED[i]
        x0 = x0 + jnp.uint32(ka)
        x1 = x1 + jnp.uint32(kb) + jnp.uint32(i + 1)
    return x0 ^ x1


def _sample_kernel(x_ref, o_ref, max_ref, idx_ref):
    b = pl.program_id(0)
    nb = pl.num_programs(0)

    @pl.when(b == 0)
    def _init():
        max_ref[...] = jnp.full_like(max_ref[...], -jnp.inf)
        idx_ref[...] = jnp.zeros_like(idx_ref[...])

    acc_m = max_ref[...]
    acc_i = idx_ref[...]
    for c in range(_BLK // _CHUNK):
        shp = (_B, _CHUNK)
        col = (jax.lax.broadcasted_iota(jnp.int32, shp, 1)
               + (b * _BLK + c * _CHUNK))
        row = jax.lax.broadcasted_iota(jnp.uint32, shp, 0)
        flat = row * jnp.uint32(_V) + col.astype(jnp.uint32)

        bits = _threefry_bits(flat)

        fbits = (bits >> jnp.uint32(9)) | jnp.uint32(0x3F800000)
        u = jax.lax.bitcast_convert_type(fbits, jnp.float32) - 1.0
        u = jnp.maximum(u, jnp.float32(1e-20))
        gumbel = -jnp.log(-jnp.log(u))

        pert = x_ref[:, c * _CHUNK:(c + 1) * _CHUNK] + gumbel
        pert = jnp.where(col < _V, pert, -jnp.inf)

        for s in range(_CHUNK // 128):
            tile = pert[:, s * 128:(s + 1) * 128]
            itile = col[:, s * 128:(s + 1) * 128]
            upd = tile > acc_m
            acc_i = jnp.where(upd, itile, acc_i)
            acc_m = jnp.maximum(acc_m, tile)
    max_ref[...] = acc_m
    idx_ref[...] = acc_i

    @pl.when(b == nb - 1)
    def _done():
        m = jnp.max(acc_m, axis=1, keepdims=True)
        cand = jnp.where(acc_m == m, acc_i, jnp.int32(0x7FFFFFFF))
        o_ref[...] = jnp.min(cand, axis=1, keepdims=True)


def kernel(logits):
    out = pl.pallas_call(
        _sample_kernel,
        grid=(pl.cdiv(_V, _BLK),),
        in_specs=[pl.BlockSpec((_B, _BLK), lambda b: (0, b))],
        out_specs=pl.BlockSpec((_B, 1), lambda b: (0, 0)),
        out_shape=jax.ShapeDtypeStruct((_B, 1), jnp.int32),
        scratch_shapes=[
            pltpu.VMEM((_B, 128), jnp.float32),
            pltpu.VMEM((_B, 128), jnp.int32),
        ],
    )(logits)
    return out[:, 0].astype(jnp.int64)
